# baseline (speedup 1.0000x reference)
import functools

import numpy as np
import jax
import jax.numpy as jnp
from jax import lax
from jax.experimental import pallas as pl
from jax.experimental.pallas import tpu as pltpu
from jax.experimental.pallas import tpu_sc as plsc

B = 16384
N_CONT = 13
N_CAT = 26
V = 100001
D = 32
N_OUT = N_CONT + N_CAT
EPS = 1e-5

NW = 32
BPT = B // NW
NB = 64
NR = NB * N_CAT
NCHUNK = BPT // NB
NBT = 512


def _stats_body(x_ref, g_ref, beta_ref, o_ref):
    x = x_ref[...]
    mean = jnp.mean(x, axis=0)
    var = jnp.mean((x - mean) ** 2, axis=0)
    scale = g_ref[...] * lax.rsqrt(var + EPS)
    shift = beta_ref[...] - mean * scale
    o_ref[...] = jnp.stack([scale, shift])


def _asm_body(x_ref, ss_ref, w_ref, b_ref, cat_ref, o_ref):
    xn = x_ref[...] * ss_ref[0][None, :] + ss_ref[1][None, :]
    xc = w_ref[...][None] * xn[:, :, None] + b_ref[...][None]
    xcat = cat_ref[...].reshape(NBT, N_CAT, D)
    o_ref[...] = jnp.concatenate([xc, xcat], axis=1)


_mesh = plsc.VectorSubcoreMesh(core_axis_name="c", subcore_axis_name="s")


@functools.partial(
    pl.kernel,
    mesh=_mesh,
    compiler_params=pltpu.CompilerParams(
        needs_layout_passes=False, use_tc_tiling_on_sc=False),
    out_type=jax.ShapeDtypeStruct((B * N_CAT, D), jnp.float32),
    scratch_types=[
        pltpu.VMEM((NR,), jnp.int32),
        pltpu.VMEM((NR, D), jnp.float32),
        pltpu.SemaphoreType.DMA,
    ],
)
def _sc_gather(cat_hbm, tab_hbm, out_hbm, idx1, catbuf, sem):
    wid = lax.axis_index("s") * 2 + lax.axis_index("c")
    out2 = out_hbm
    lanes = lax.iota(jnp.int32, 16)
    offs = [((lanes + g * 16) % N_CAT) * V for g in range(13)]

    def chunk_body(ci, carry):
        base = wid * BPT + ci * NB
        pltpu.sync_copy(cat_hbm.at[pl.ds(base * N_CAT, NR)], idx1)
        for g in range(NR // 16):
            sl = pl.ds(g * 16, 16)
            idx1[sl] = idx1[sl] + offs[g % 13]
        pltpu.async_copy(tab_hbm.at[idx1], catbuf, sem).wait()
        pltpu.sync_copy(catbuf, out2.at[pl.ds(base * N_CAT, NR)])
        return carry

    lax.fori_loop(0, NCHUNK, chunk_body, 0)


def kernel(x, categorical, gamma, beta, W, b, tables):
    ss = pl.pallas_call(
        _stats_body,
        out_shape=jax.ShapeDtypeStruct((2, N_CONT), jnp.float32),
    )(x, gamma, beta)
    cf = categorical.astype(jnp.int32).reshape(-1)
    tf = tables.reshape(N_CAT * V, D)
    xcat = _sc_gather(cf, tf)
    out = pl.pallas_call(
        _asm_body,
        grid=(B // NBT,),
        in_specs=[
            pl.BlockSpec((NBT, N_CONT), lambda i: (i, 0)),
            pl.BlockSpec((2, N_CONT), lambda i: (0, 0)),
            pl.BlockSpec((N_CONT, D), lambda i: (0, 0)),
            pl.BlockSpec((N_CONT, D), lambda i: (0, 0)),
            pl.BlockSpec((NBT * N_CAT, D), lambda i: (i, 0)),
        ],
        out_specs=pl.BlockSpec((NBT, N_OUT, D), lambda i: (i, 0, 0)),
        out_shape=jax.ShapeDtypeStruct((B, N_OUT, D), jnp.float32),
    )(x, ss, W, b, xcat)
    return out


# speedup vs baseline: 1.0122x; 1.0122x over previous
import functools

import jax
import jax.numpy as jnp
from jax import lax
from jax.experimental import pallas as pl
from jax.experimental.pallas import tpu as pltpu
from jax.experimental.pallas import tpu_sc as plsc

B = 16384
N_CONT = 13
N_CAT = 26
V = 100001
D = 32
N_OUT = N_CONT + N_CAT
EPS = 1e-5

NW = 32
BPT = B // NW
NB = 128
NR = NB * N_CAT
NCHUNK = BPT // NB
NBT = 512
CATL = 128
CAT_ROWS = B * N_CAT * D // CATL


def _stats_body(x_ref, g_ref, beta_ref, o_ref):
    x = x_ref[...]
    mean = jnp.mean(x, axis=0)
    var = jnp.mean((x - mean) ** 2, axis=0)
    scale = g_ref[...] * lax.rsqrt(var + EPS)
    shift = beta_ref[...] - mean * scale
    o_ref[...] = jnp.stack([scale, shift])


def _cont_body(x_ref, ss_ref, w_ref, b_ref, o_ref):
    xn = x_ref[...] * ss_ref[0][None, :] + ss_ref[1][None, :]
    o_ref[...] = w_ref[...][None] * xn[:, :, None] + b_ref[...][None]


_mesh = plsc.VectorSubcoreMesh(core_axis_name="c", subcore_axis_name="s")


@functools.partial(
    pl.kernel,
    mesh=_mesh,
    compiler_params=pltpu.CompilerParams(
        needs_layout_passes=False, use_tc_tiling_on_sc=False),
    out_type=jax.ShapeDtypeStruct((B * N_CAT, D), jnp.float32),
    scratch_types=[
        pltpu.VMEM((NR,), jnp.int32),
        pltpu.VMEM((NR, D), jnp.float32),
        pltpu.SemaphoreType.DMA,
    ],
)
def _sc_gather(ids_hbm, tab_hbm, out_hbm, idx1, catbuf, sem):
    wid = lax.axis_index("s") * 2 + lax.axis_index("c")
    lanes = lax.iota(jnp.int32, 16)
    offs = [((lanes + g * 16) % N_CAT) * V for g in range(13)]

    def chunk_body(ci, carry):
        base = wid * BPT + ci * NB
        pltpu.sync_copy(ids_hbm.at[pl.ds(base * N_CAT, NR)], idx1)
        for g in range(NR // 16):
            sl = pl.ds(g * 16, 16)
            idx1[sl] = idx1[sl] + offs[g % 13]
        pltpu.async_copy(tab_hbm.at[idx1], catbuf, sem).wait()
        pltpu.sync_copy(catbuf, out_hbm.at[pl.ds(base * N_CAT, NR)])
        return carry

    lax.fori_loop(0, NCHUNK, chunk_body, 0)


def kernel(x, categorical, gamma, beta, W, b, tables):
    ids = categorical.astype(jnp.int32).reshape(-1)
    ss = pl.pallas_call(
        _stats_body,
        out_shape=jax.ShapeDtypeStruct((2, N_CONT), jnp.float32),
    )(x, gamma, beta)
    tf = tables.reshape(N_CAT * V, D)
    xcat = _sc_gather(ids, tf)
    xcont = pl.pallas_call(
        _cont_body,
        grid=(B // NBT,),
        in_specs=[
            pl.BlockSpec((NBT, N_CONT), lambda i: (i, 0)),
            pl.BlockSpec((2, N_CONT), lambda i: (0, 0)),
            pl.BlockSpec((N_CONT, D), lambda i: (0, 0)),
            pl.BlockSpec((N_CONT, D), lambda i: (0, 0)),
        ],
        out_specs=pl.BlockSpec((NBT, N_CONT, D), lambda i: (i, 0, 0)),
        out_shape=jax.ShapeDtypeStruct((B, N_CONT, D), jnp.float32),
    )(x, ss, W, b)
    return jnp.concatenate([xcont, xcat.reshape(B, N_CAT, D)], axis=1)


# speedup vs baseline: 2.3985x; 2.3696x over previous
import functools

import jax
import jax.numpy as jnp
from jax import lax
from jax.experimental import pallas as pl
from jax.experimental.pallas import tpu as pltpu
from jax.experimental.pallas import tpu_sc as plsc

B = 16384
N_CONT = 13
N_CAT = 26
V = 100001
D = 32
N_OUT = N_CONT + N_CAT
EPS = 1e-5

NW = 32
BPT = B // NW
NB = 64
NR = NB * N_CAT
NCHUNK = BPT // NB
NBT = 512
CATL = 128
CAT_ROWS = B * N_CAT * D // CATL


def _stats_body(x_ref, g_ref, beta_ref, o_ref):
    x = x_ref[...]
    mean = jnp.mean(x, axis=0)
    var = jnp.mean((x - mean) ** 2, axis=0)
    scale = g_ref[...] * lax.rsqrt(var + EPS)
    shift = beta_ref[...] - mean * scale
    o_ref[...] = jnp.stack([scale, shift])


def _cont_body(x_ref, ss_ref, w_ref, b_ref, o_ref):
    xn = x_ref[...] * ss_ref[0][None, :] + ss_ref[1][None, :]
    o_ref[...] = w_ref[...][None] * xn[:, :, None] + b_ref[...][None]


_mesh = plsc.VectorSubcoreMesh(core_axis_name="c", subcore_axis_name="s")


@functools.partial(
    pl.kernel,
    mesh=_mesh,
    compiler_params=pltpu.CompilerParams(
        needs_layout_passes=False, use_tc_tiling_on_sc=False),
    out_type=jax.ShapeDtypeStruct((B * N_CAT, D), jnp.float32),
    name="sc_embedding_gather",
    scratch_types=[
        pltpu.VMEM((NB, N_CAT), jnp.int32),
        pltpu.VMEM((N_CAT, NB), jnp.int32),
        pltpu.VMEM((NR, D), jnp.float32),
        pltpu.VMEM((NR, D), jnp.float32),
        pltpu.SemaphoreType.DMA,
    ],
)
def _sc_gather(cat2_hbm, tab3_hbm, out_hbm, idbuf, idxf, cat_fm, cat_bm, sem):
    wid = lax.axis_index("s") * 2 + lax.axis_index("c")
    lanes = lax.iota(jnp.int32, 16)

    def chunk_body(ci, carry):
        base = wid * BPT + ci * NB
        pltpu.sync_copy(cat2_hbm.at[pl.ds(base, NB)], idbuf)
        for f in range(N_CAT):
            fv = jnp.full((16,), f, jnp.int32)
            for k in range(NB // 16):
                b16 = lanes + k * 16
                v = plsc.load_gather(idbuf, [b16, fv])
                plsc.store_scatter(idxf, [fv, b16], v)
        cps = []
        for f in range(N_CAT):
            cps.append(pltpu.async_copy(
                tab3_hbm.at[f].at[idxf.at[f]],
                cat_fm.at[pl.ds(f * NB, NB)],
                sem))
        for cp in cps:
            cp.wait()
        def rint_body(b, carry2):
            for f in range(N_CAT):
                sv = jnp.full((16,), f * NB + b, jnp.int32)
                dv = jnp.full((16,), b * N_CAT + f, jnp.int32)
                v0 = plsc.load_gather(cat_fm, [sv, lanes])
                v1 = plsc.load_gather(cat_fm, [sv, lanes + 16])
                plsc.store_scatter(cat_bm, [dv, lanes], v0)
                plsc.store_scatter(cat_bm, [dv, lanes + 16], v1)
            return carry2

        lax.fori_loop(0, NB, rint_body, 0)
        pltpu.sync_copy(cat_bm, out_hbm.at[pl.ds(base * N_CAT, NR)])
        return carry

    lax.fori_loop(0, NCHUNK, chunk_body, 0)


def kernel(x, categorical, gamma, beta, W, b, tables):
    ss = pl.pallas_call(
        _stats_body,
        out_shape=jax.ShapeDtypeStruct((2, N_CONT), jnp.float32),
    )(x, gamma, beta)
    xcat = _sc_gather(categorical.astype(jnp.int32), tables)
    xcont = pl.pallas_call(
        _cont_body,
        grid=(B // NBT,),
        in_specs=[
            pl.BlockSpec((NBT, N_CONT), lambda i: (i, 0)),
            pl.BlockSpec((2, N_CONT), lambda i: (0, 0)),
            pl.BlockSpec((N_CONT, D), lambda i: (0, 0)),
            pl.BlockSpec((N_CONT, D), lambda i: (0, 0)),
        ],
        out_specs=pl.BlockSpec((NBT, N_CONT, D), lambda i: (i, 0, 0)),
        out_shape=jax.ShapeDtypeStruct((B, N_CONT, D), jnp.float32),
    )(x, ss, W, b)
    return jnp.concatenate([xcont, xcat.reshape(B, N_CAT, D)], axis=1)


# speedup vs baseline: 5.4077x; 2.2546x over previous
import functools

import jax
import jax.numpy as jnp
from jax import lax
from jax.experimental import pallas as pl
from jax.experimental.pallas import tpu as pltpu
from jax.experimental.pallas import tpu_sc as plsc

B = 16384
N_CONT = 13
N_CAT = 26
V = 100001
D = 32
N_OUT = N_CONT + N_CAT
EPS = 1e-5

NW = 32
BPT = B // NW
NB = 64
NR = NB * N_CAT
NCHUNK = BPT // NB
NBT = 512
CATL = 128
CAT_ROWS = B * N_CAT * D // CATL


def _stats_body(x_ref, g_ref, beta_ref, o_ref):
    x = x_ref[...]
    mean = jnp.mean(x, axis=0)
    var = jnp.mean((x - mean) ** 2, axis=0)
    scale = g_ref[...] * lax.rsqrt(var + EPS)
    shift = beta_ref[...] - mean * scale
    o_ref[...] = jnp.stack([scale, shift])


def _cont_body(x_ref, ss_ref, w_ref, b_ref, o_ref):
    xn = x_ref[...] * ss_ref[0][None, :] + ss_ref[1][None, :]
    o_ref[...] = w_ref[...][None] * xn[:, :, None] + b_ref[...][None]


_mesh = plsc.VectorSubcoreMesh(core_axis_name="c", subcore_axis_name="s")


@functools.partial(
    pl.kernel,
    mesh=_mesh,
    compiler_params=pltpu.CompilerParams(
        needs_layout_passes=False, use_tc_tiling_on_sc=False),
    out_type=jax.ShapeDtypeStruct((B * N_CAT, D), jnp.float32),
    name="sc_embedding_gather",
    scratch_types=[
        pltpu.VMEM((NB, N_CAT), jnp.int32),
        pltpu.VMEM((N_CAT, NB), jnp.int32),
        pltpu.VMEM((NR, D), jnp.float32),
        pltpu.VMEM((NR, D), jnp.float32),
        pltpu.SemaphoreType.DMA,
    ],
)
def _sc_gather(cat2_hbm, *rest):
    tabs = rest[:N_CAT]
    out_hbm = rest[N_CAT]
    idbuf, idxf, cat_fm, cat_bm, sem = rest[N_CAT + 1:]
    wid = lax.axis_index("s") * 2 + lax.axis_index("c")
    lanes = lax.iota(jnp.int32, 16)

    def chunk_body(ci, carry):
        base = wid * BPT + ci * NB
        pltpu.sync_copy(cat2_hbm.at[pl.ds(base, NB)], idbuf)
        for f in range(N_CAT):
            fv = jnp.full((16,), f, jnp.int32)
            for k in range(NB // 16):
                b16 = lanes + k * 16
                v = plsc.load_gather(idbuf, [b16, fv])
                plsc.store_scatter(idxf, [fv, b16], v)
        cps = []
        for f in range(N_CAT):
            cps.append(pltpu.async_copy(
                tabs[f].at[idxf.at[f]],
                cat_fm.at[pl.ds(f * NB, NB)],
                sem))
        for cp in cps:
            cp.wait()
        def rint_body(b, carry2):
            for f in range(N_CAT):
                sv = jnp.full((16,), f * NB + b, jnp.int32)
                dv = jnp.full((16,), b * N_CAT + f, jnp.int32)
                v0 = plsc.load_gather(cat_fm, [sv, lanes])
                v1 = plsc.load_gather(cat_fm, [sv, lanes + 16])
                plsc.store_scatter(cat_bm, [dv, lanes], v0)
                plsc.store_scatter(cat_bm, [dv, lanes + 16], v1)
            return carry2

        lax.fori_loop(0, NB, rint_body, 0)
        pltpu.sync_copy(cat_bm, out_hbm.at[pl.ds(base * N_CAT, NR)])
        return carry

    lax.fori_loop(0, NCHUNK, chunk_body, 0)


def kernel(x, categorical, gamma, beta, W, b, tables):
    ss = pl.pallas_call(
        _stats_body,
        out_shape=jax.ShapeDtypeStruct((2, N_CONT), jnp.float32),
    )(x, gamma, beta)
    xcat = _sc_gather(categorical.astype(jnp.int32),
                      *[tables[f] for f in range(N_CAT)])
    xcont = pl.pallas_call(
        _cont_body,
        grid=(B // NBT,),
        in_specs=[
            pl.BlockSpec((NBT, N_CONT), lambda i: (i, 0)),
            pl.BlockSpec((2, N_CONT), lambda i: (0, 0)),
            pl.BlockSpec((N_CONT, D), lambda i: (0, 0)),
            pl.BlockSpec((N_CONT, D), lambda i: (0, 0)),
        ],
        out_specs=pl.BlockSpec((NBT, N_CONT, D), lambda i: (i, 0, 0)),
        out_shape=jax.ShapeDtypeStruct((B, N_CONT, D), jnp.float32),
    )(x, ss, W, b)
    return jnp.concatenate([xcont, xcat.reshape(B, N_CAT, D)], axis=1)
